# baseline (speedup 1.0000x reference)
import functools

import jax
import jax.numpy as jnp
from jax import lax
from jax.experimental import pallas as pl
from jax.experimental.pallas import tpu as pltpu
from jax.experimental.pallas import tpu_sc as plsc

N = 10000
E = 320000
D = 128
H = D // 2
G = 64
RD = 182

NC = 2
NS = 16
K = 128

N_PAD = 10240
E_PAD = ((E + 2 * NC * NS * K - 1) // (2 * NC * NS * K)) * (2 * NC * NS * K)
PER_CORE = E_PAD // NC
PER_TILE = PER_CORE // NS
NCHUNK = PER_TILE // K
TOT_CHUNKS = E_PAD // K
C0 = 160
C1 = TOT_CHUNKS // NS - C0
MAXC = max(C0, C1)
STAGE_CHUNKS = NS * C0 + (NS - 1) * C1 + MAXC
ROWS_PER_TILE = N_PAD // NS

_mesh = plsc.VectorSubcoreMesh(
    core_axis_name="c", subcore_axis_name="s", num_cores=NC, num_subcores=NS
)
_sc_params = pltpu.CompilerParams(use_tc_tiling_on_sc=False)
_sc_params_nl = pltpu.CompilerParams(
    use_tc_tiling_on_sc=False, needs_layout_passes=False
)


@functools.partial(
    pl.kernel,
    out_type=jax.ShapeDtypeStruct((NC, NS, N_PAD), jnp.float32),
    mesh=_mesh,
    scratch_types=[
        pltpu.VMEM((PER_TILE,), jnp.int32),
        pltpu.VMEM((N_PAD,), jnp.float32),
    ],
    compiler_params=_sc_params_nl,
)
def _sc_degree(dst_hbm, deg_out, dst_v, deg_local):
    cid = lax.axis_index("c")
    sid = lax.axis_index("s")

    def fill_zero(i, _):
        deg_local[pl.ds(i * 16, 16)] = jnp.zeros((16,), jnp.float32)
        return 0
    lax.fori_loop(0, N_PAD // 16, fill_zero, 0)

    tile_base = cid * PER_CORE + sid * PER_TILE
    pltpu.sync_copy(dst_hbm.at[pl.ds(tile_base, PER_TILE)], dst_v)

    def cnt(i, _):
        v = dst_v[pl.ds(i * 16, 16)]
        c, last = plsc.scan_count(v)
        plsc.addupdate_scatter(deg_local, [v], c.astype(jnp.float32), mask=last)
        return 0
    lax.fori_loop(0, PER_TILE // 16, cnt, 0)

    pltpu.sync_copy(deg_local, deg_out.at[cid, sid])


@functools.partial(
    pl.kernel,
    out_type=(
        jax.ShapeDtypeStruct((NC, N_PAD, H), jnp.float32),
        jax.ShapeDtypeStruct((NC, N_PAD, H), jnp.float32),
    ),
    mesh=_mesh,
    scratch_types=[
        pltpu.VMEM((MAXC, K), jnp.int32),
        pltpu.VMEM((MAXC, K), jnp.int32),
        pltpu.VMEM((K, H), jnp.float32),
        pltpu.VMEM((K, H), jnp.float32),
        pltpu.VMEM((ROWS_PER_TILE // 8, H), jnp.float32),
        pltpu.VMEM_SHARED((N_PAD, H), jnp.float32),
        pltpu.SemaphoreType.DMA,
        pltpu.SemaphoreType.DMA,
    ],
    compiler_params=_sc_params,
)
def _sc_aggregate(hs_lo, hs_hi, src_hbm, dst_hbm, out_lo, out_hi,
                  src_all, dst_all, rows0, rows1, zslab, acc, sem0, sem1):
    cid = lax.axis_index("c")
    sid = lax.axis_index("s")

    def fill_zero(i, _):
        zslab[i, pl.ds(0, 16)] = jnp.zeros((16,), jnp.float32)
        zslab[i, pl.ds(16, 16)] = jnp.zeros((16,), jnp.float32)
        zslab[i, pl.ds(32, 16)] = jnp.zeros((16,), jnp.float32)
        zslab[i, pl.ds(48, 16)] = jnp.zeros((16,), jnp.float32)
        return 0
    lax.fori_loop(0, ROWS_PER_TILE // 8, fill_zero, 0)

    chunk0 = jnp.where(cid == 0, sid * C0, NS * C0 + sid * C1)
    my_pairs = jnp.where(cid == 0, C0 // 2, C1 // 2)
    pltpu.sync_copy(src_hbm.at[pl.ds(chunk0, MAXC)], src_all)
    pltpu.sync_copy(dst_hbm.at[pl.ds(chunk0, MAXC)], dst_all)
    my_rows = pl.ds(sid * ROWS_PER_TILE, ROWS_PER_TILE)

    for table, out in ((hs_lo, out_lo), (hs_hi, out_hi)):
        for r in range(8):
            pltpu.sync_copy(
                zslab,
                acc.at[pl.ds(sid * ROWS_PER_TILE + r * (ROWS_PER_TILE // 8),
                             ROWS_PER_TILE // 8)],
            )
        plsc.subcore_barrier()

        @pl.when(my_pairs > 0)
        def _():
            pltpu.async_copy(table.at[src_all.at[0]], rows0, sem0)

        def pair(p, _):
            @pl.when(p < my_pairs)
            def _():
                c0 = 2 * p
                pltpu.async_copy(table.at[src_all.at[c0 + 1]], rows1, sem1)
                pltpu.make_async_copy(table.at[src_all.at[c0]], rows0, sem0).wait()
                pltpu.sync_copy(rows0, acc.at[dst_all.at[c0]], add=True)

                @pl.when(p + 1 < my_pairs)
                def _():
                    pltpu.async_copy(table.at[src_all.at[c0 + 2]], rows0, sem0)

                pltpu.make_async_copy(table.at[src_all.at[c0 + 1]], rows1, sem1).wait()
                pltpu.sync_copy(rows1, acc.at[dst_all.at[c0 + 1]], add=True)
            return 0
        lax.fori_loop(0, MAXC // 2, pair, 0)

        plsc.subcore_barrier()
        pltpu.sync_copy(acc.at[my_rows], out.at[cid, my_rows])
        plsc.subcore_barrier()


def _tc_start_body(degp_ref, x_ref, w_ref, dinv_ref, lo_ref, hi_ref):
    parts = jnp.reshape(degp_ref[...], (NC * NS, N_PAD))
    deg = jnp.sum(parts, axis=0)[0:N, None] + 1.0
    dinv = lax.rsqrt(deg)
    dinv_ref[...] = dinv
    h = jnp.dot(x_ref[...], w_ref[...], preferred_element_type=jnp.float32)
    hs = h * dinv
    lo_ref[...] = hs[:, 0:H]
    hi_ref[...] = hs[:, H:D]


def _bn_relu(p_lo, p_hi, hs_lo, hs_hi, dinv, b, g, be):
    a_lo = p_lo[0, 0:N, :] + p_lo[1, 0:N, :] + hs_lo
    a_hi = p_hi[0, 0:N, :] + p_hi[1, 0:N, :] + hs_hi
    o = jnp.concatenate([a_lo, a_hi], axis=1) * dinv + b
    m = jnp.mean(o, axis=0, keepdims=True)
    v = jnp.mean((o - m) * (o - m), axis=0, keepdims=True)
    y = g * (o - m) * lax.rsqrt(v + 1e-5) + be
    return jnp.maximum(y, 0.0)


def _tc_mid_body(plo_ref, phi_ref, hlo_ref, hhi_ref, dinv_ref, b_ref, g_ref,
                 be_ref, w_ref, lo_ref, hi_ref):
    dinv = dinv_ref[...]
    y = _bn_relu(plo_ref[...], phi_ref[...], hlo_ref[...], hhi_ref[...],
                 dinv, b_ref[...], g_ref[...], be_ref[...])
    hs = jnp.dot(y, w_ref[...], preferred_element_type=jnp.float32) * dinv
    lo_ref[...] = hs[:, 0:H]
    hi_ref[...] = hs[:, H:D]


def _tc_final_body(plo_ref, phi_ref, hlo_ref, hhi_ref, dinv_ref, b_ref, g_ref,
                   be_ref, batch_ref, rd_ref, w1a_ref, w1b_ref, bo1_ref,
                   w2_ref, bo2_ref, w3_ref, bo3_ref, out_ref):
    y = _bn_relu(plo_ref[...], phi_ref[...], hlo_ref[...], hhi_ref[...],
                 dinv_ref[...], b_ref[...], g_ref[...], be_ref[...])

    gi = lax.broadcasted_iota(jnp.int32, (G, N), 0)
    selt = (gi == batch_ref[...]).astype(jnp.float32)
    sums = jnp.dot(selt, y, preferred_element_type=jnp.float32)
    cnt = jnp.sum(selt, axis=1, keepdims=True)
    pooled = sums / jnp.maximum(cnt, 1.0)

    z = jnp.dot(pooled, w1a_ref[...], preferred_element_type=jnp.float32)
    z = z + jnp.dot(rd_ref[...], w1b_ref[...], preferred_element_type=jnp.float32)
    z = jnp.maximum(z + bo1_ref[...], 0.0)
    z = jnp.maximum(jnp.dot(z, w2_ref[...], preferred_element_type=jnp.float32) + bo2_ref[...], 0.0)
    out_ref[...] = jnp.dot(z, w3_ref[...], preferred_element_type=jnp.float32) + bo3_ref[...]


_tc_start = pl.pallas_call(
    _tc_start_body,
    out_shape=[
        jax.ShapeDtypeStruct((N, 1), jnp.float32),
        jax.ShapeDtypeStruct((N, H), jnp.float32),
        jax.ShapeDtypeStruct((N, H), jnp.float32),
    ],
)

_tc_mid = pl.pallas_call(
    _tc_mid_body,
    out_shape=[
        jax.ShapeDtypeStruct((N, H), jnp.float32),
        jax.ShapeDtypeStruct((N, H), jnp.float32),
    ],
)

_tc_final = pl.pallas_call(
    _tc_final_body,
    out_shape=jax.ShapeDtypeStruct((G, 1), jnp.float32),
)


@jax.jit
def kernel(x, edge_index, batch, rdkit_vec, W0, b0, g0, be0, W1, b1, g1, be1,
           Wo1, bo1, Wo2, bo2, Wo3, bo3):
    src = edge_index[0]
    dst = edge_index[1]
    pad = STAGE_CHUNKS * K - E
    src_p = jnp.concatenate([src, jnp.zeros((pad,), jnp.int32)])
    dummy_dst = N + jnp.arange(pad, dtype=jnp.int32) % jnp.int32(N_PAD - N)
    dst_p = jnp.concatenate([dst, dummy_dst])
    src_2d = src_p.reshape(STAGE_CHUNKS, K)
    dst_2d = dst_p.reshape(STAGE_CHUNKS, K)

    deg_parts = _sc_degree(dst_p[:E_PAD])
    dinv, hs0_lo, hs0_hi = _tc_start(deg_parts, x, W0)

    p0_lo, p0_hi = _sc_aggregate(hs0_lo, hs0_hi, src_2d, dst_2d)
    hs1_lo, hs1_hi = _tc_mid(p0_lo, p0_hi, hs0_lo, hs0_hi, dinv,
                             b0.reshape(1, D), g0.reshape(1, D),
                             be0.reshape(1, D), W1)

    p1_lo, p1_hi = _sc_aggregate(hs1_lo, hs1_hi, src_2d, dst_2d)
    out = _tc_final(p1_lo, p1_hi, hs1_lo, hs1_hi, dinv,
                    b1.reshape(1, D), g1.reshape(1, D), be1.reshape(1, D),
                    batch.reshape(1, N), rdkit_vec,
                    Wo1[:D], Wo1[D:], bo1.reshape(1, -1), Wo2,
                    bo2.reshape(1, -1), Wo3, bo3.reshape(1, 1))
    return out.reshape(-1)


# speedup vs baseline: 1.3046x; 1.3046x over previous
import functools

import jax
import jax.numpy as jnp
from jax import lax
from jax.experimental import pallas as pl
from jax.experimental.pallas import tpu as pltpu
from jax.experimental.pallas import tpu_sc as plsc

N = 10000
E = 320000
D = 128
H = D // 2
G = 64
RD = 182

NC = 2
NS = 16
K = 128

N_PAD = 10240
E_PAD = ((E + 2 * NC * NS * K - 1) // (2 * NC * NS * K)) * (2 * NC * NS * K)
PER_CORE = E_PAD // NC
PER_TILE = PER_CORE // NS
NCHUNK = PER_TILE // K
TOT_CHUNKS = E_PAD // K
C0 = 116
C1 = TOT_CHUNKS // NS - C0
MAXC = max(C0, C1)
STAGE_CHUNKS = NS * C0 + (NS - 1) * C1 + MAXC
ROWS_PER_TILE = N_PAD // NS

_mesh = plsc.VectorSubcoreMesh(
    core_axis_name="c", subcore_axis_name="s", num_cores=NC, num_subcores=NS
)
_sc_params = pltpu.CompilerParams(use_tc_tiling_on_sc=False)
_sc_params_nl = pltpu.CompilerParams(
    use_tc_tiling_on_sc=False, needs_layout_passes=False
)


@functools.partial(
    pl.kernel,
    out_type=jax.ShapeDtypeStruct((NC, NS, N_PAD), jnp.float32),
    mesh=_mesh,
    scratch_types=[
        pltpu.VMEM((PER_TILE,), jnp.int32),
        pltpu.VMEM((N_PAD,), jnp.float32),
    ],
    compiler_params=_sc_params_nl,
)
def _sc_degree(dst_hbm, deg_out, dst_v, deg_local):
    cid = lax.axis_index("c")
    sid = lax.axis_index("s")

    def fill_zero(i, _):
        deg_local[pl.ds(i * 16, 16)] = jnp.zeros((16,), jnp.float32)
        return 0
    lax.fori_loop(0, N_PAD // 16, fill_zero, 0)

    tile_base = cid * PER_CORE + sid * PER_TILE
    pltpu.sync_copy(dst_hbm.at[pl.ds(tile_base, PER_TILE)], dst_v)

    def cnt(i, _):
        v = dst_v[pl.ds(i * 16, 16)]
        c, last = plsc.scan_count(v)
        plsc.addupdate_scatter(deg_local, [v], c.astype(jnp.float32), mask=last)
        return 0
    lax.fori_loop(0, PER_TILE // 16, cnt, 0)

    pltpu.sync_copy(deg_local, deg_out.at[cid, sid])


@functools.partial(
    pl.kernel,
    out_type=(
        jax.ShapeDtypeStruct((NC, N_PAD, H), jnp.float32),
        jax.ShapeDtypeStruct((NC, N_PAD, H), jnp.float32),
    ),
    mesh=_mesh,
    scratch_types=[
        pltpu.VMEM((MAXC, K), jnp.int32),
        pltpu.VMEM((MAXC, K), jnp.int32),
        [pltpu.VMEM((K, H), jnp.float32)] * 4,
        pltpu.VMEM((ROWS_PER_TILE // 8, H), jnp.float32),
        pltpu.VMEM_SHARED((N_PAD, H), jnp.float32),
        [pltpu.SemaphoreType.DMA] * 2,
    ],
    compiler_params=_sc_params,
)
def _sc_aggregate(hs_lo, hs_hi, src_hbm, dst_hbm, out_lo, out_hi,
                  src_all, dst_all, rows, zslab, acc, sem_g):
    cid = lax.axis_index("c")
    sid = lax.axis_index("s")

    def fill_zero(i, _):
        zslab[i, pl.ds(0, 16)] = jnp.zeros((16,), jnp.float32)
        zslab[i, pl.ds(16, 16)] = jnp.zeros((16,), jnp.float32)
        zslab[i, pl.ds(32, 16)] = jnp.zeros((16,), jnp.float32)
        zslab[i, pl.ds(48, 16)] = jnp.zeros((16,), jnp.float32)
        return 0
    lax.fori_loop(0, ROWS_PER_TILE // 8, fill_zero, 0)

    chunk0 = jnp.where(cid == 0, sid * C0, NS * C0 + sid * C1)
    my_quads = jnp.where(cid == 0, C0 // 4, C1 // 4)
    pltpu.sync_copy(src_hbm.at[pl.ds(chunk0, MAXC)], src_all)
    pltpu.sync_copy(dst_hbm.at[pl.ds(chunk0, MAXC)], dst_all)
    my_rows = pl.ds(sid * ROWS_PER_TILE, ROWS_PER_TILE)

    for table, out in ((hs_lo, out_lo), (hs_hi, out_hi)):
        for r in range(8):
            pltpu.sync_copy(
                zslab,
                acc.at[pl.ds(sid * ROWS_PER_TILE + r * (ROWS_PER_TILE // 8),
                             ROWS_PER_TILE // 8)],
            )
        plsc.subcore_barrier()

        @pl.when(my_quads > 0)
        def _():
            pltpu.async_copy(table.at[src_all.at[0]], rows[0], sem_g[0])
            pltpu.async_copy(table.at[src_all.at[1]], rows[1], sem_g[1])

        def quad(q, _):
            @pl.when(q < my_quads)
            def _():
                for j in range(4):
                    c = 4 * q + j
                    b = j
                    bn = (j + 2) % 4
                    p = j % 2
                    pltpu.make_async_copy(table.at[src_all.at[c]], rows[b], sem_g[p]).wait()
                    pltpu.sync_copy(rows[b], acc.at[dst_all.at[c]], add=True)
                    if j < 2:
                        pltpu.async_copy(table.at[src_all.at[c + 2]], rows[bn], sem_g[p])
                    else:
                        @pl.when(q + 1 < my_quads)
                        def _():
                            pltpu.async_copy(table.at[src_all.at[c + 2]], rows[bn], sem_g[p])
            return 0
        lax.fori_loop(0, MAXC // 4, quad, 0)

        plsc.subcore_barrier()
        pltpu.sync_copy(acc.at[my_rows], out.at[cid, my_rows])
        plsc.subcore_barrier()


def _tc_start_body(degp_ref, x_ref, w_ref, dinv_ref, lo_ref, hi_ref):
    parts = jnp.reshape(degp_ref[...], (NC * NS, N_PAD))
    deg = jnp.sum(parts, axis=0)[0:N, None] + 1.0
    dinv = lax.rsqrt(deg)
    dinv_ref[...] = dinv
    h = jnp.dot(x_ref[...], w_ref[...], preferred_element_type=jnp.float32)
    hs = h * dinv
    lo_ref[...] = hs[:, 0:H]
    hi_ref[...] = hs[:, H:D]


def _bn_relu(p_lo, p_hi, hs_lo, hs_hi, dinv, b, g, be):
    a_lo = p_lo[0, 0:N, :] + p_lo[1, 0:N, :] + hs_lo
    a_hi = p_hi[0, 0:N, :] + p_hi[1, 0:N, :] + hs_hi
    o = jnp.concatenate([a_lo, a_hi], axis=1) * dinv + b
    m = jnp.mean(o, axis=0, keepdims=True)
    v = jnp.mean((o - m) * (o - m), axis=0, keepdims=True)
    y = g * (o - m) * lax.rsqrt(v + 1e-5) + be
    return jnp.maximum(y, 0.0)


def _tc_mid_body(plo_ref, phi_ref, hlo_ref, hhi_ref, dinv_ref, b_ref, g_ref,
                 be_ref, w_ref, lo_ref, hi_ref):
    dinv = dinv_ref[...]
    y = _bn_relu(plo_ref[...], phi_ref[...], hlo_ref[...], hhi_ref[...],
                 dinv, b_ref[...], g_ref[...], be_ref[...])
    hs = jnp.dot(y, w_ref[...], preferred_element_type=jnp.float32) * dinv
    lo_ref[...] = hs[:, 0:H]
    hi_ref[...] = hs[:, H:D]


def _tc_final_body(plo_ref, phi_ref, hlo_ref, hhi_ref, dinv_ref, b_ref, g_ref,
                   be_ref, batch_ref, rd_ref, w1a_ref, w1b_ref, bo1_ref,
                   w2_ref, bo2_ref, w3_ref, bo3_ref, out_ref):
    y = _bn_relu(plo_ref[...], phi_ref[...], hlo_ref[...], hhi_ref[...],
                 dinv_ref[...], b_ref[...], g_ref[...], be_ref[...])

    gi = lax.broadcasted_iota(jnp.int32, (G, N), 0)
    selt = (gi == batch_ref[...]).astype(jnp.float32)
    sums = jnp.dot(selt, y, preferred_element_type=jnp.float32)
    cnt = jnp.sum(selt, axis=1, keepdims=True)
    pooled = sums / jnp.maximum(cnt, 1.0)

    z = jnp.dot(pooled, w1a_ref[...], preferred_element_type=jnp.float32)
    z = z + jnp.dot(rd_ref[...], w1b_ref[...], preferred_element_type=jnp.float32)
    z = jnp.maximum(z + bo1_ref[...], 0.0)
    z = jnp.maximum(jnp.dot(z, w2_ref[...], preferred_element_type=jnp.float32) + bo2_ref[...], 0.0)
    out_ref[...] = jnp.dot(z, w3_ref[...], preferred_element_type=jnp.float32) + bo3_ref[...]


_tc_start = pl.pallas_call(
    _tc_start_body,
    out_shape=[
        jax.ShapeDtypeStruct((N, 1), jnp.float32),
        jax.ShapeDtypeStruct((N, H), jnp.float32),
        jax.ShapeDtypeStruct((N, H), jnp.float32),
    ],
)

_tc_mid = pl.pallas_call(
    _tc_mid_body,
    out_shape=[
        jax.ShapeDtypeStruct((N, H), jnp.float32),
        jax.ShapeDtypeStruct((N, H), jnp.float32),
    ],
)

_tc_final = pl.pallas_call(
    _tc_final_body,
    out_shape=jax.ShapeDtypeStruct((G, 1), jnp.float32),
)


@jax.jit
def kernel(x, edge_index, batch, rdkit_vec, W0, b0, g0, be0, W1, b1, g1, be1,
           Wo1, bo1, Wo2, bo2, Wo3, bo3):
    src = edge_index[0]
    dst = edge_index[1]
    pad = STAGE_CHUNKS * K - E
    src_p = jnp.concatenate([src, jnp.zeros((pad,), jnp.int32)])
    dummy_dst = N + jnp.arange(pad, dtype=jnp.int32) % jnp.int32(N_PAD - N)
    dst_p = jnp.concatenate([dst, dummy_dst])
    src_2d = src_p.reshape(STAGE_CHUNKS, K)
    dst_2d = dst_p.reshape(STAGE_CHUNKS, K)

    deg_parts = _sc_degree(dst_p[:E_PAD])
    dinv, hs0_lo, hs0_hi = _tc_start(deg_parts, x, W0)

    p0_lo, p0_hi = _sc_aggregate(hs0_lo, hs0_hi, src_2d, dst_2d)
    hs1_lo, hs1_hi = _tc_mid(p0_lo, p0_hi, hs0_lo, hs0_hi, dinv,
                             b0.reshape(1, D), g0.reshape(1, D),
                             be0.reshape(1, D), W1)

    p1_lo, p1_hi = _sc_aggregate(hs1_lo, hs1_hi, src_2d, dst_2d)
    out = _tc_final(p1_lo, p1_hi, hs1_lo, hs1_hi, dinv,
                    b1.reshape(1, D), g1.reshape(1, D), be1.reshape(1, D),
                    batch.reshape(1, N), rdkit_vec,
                    Wo1[:D], Wo1[D:], bo1.reshape(1, -1), Wo2,
                    bo2.reshape(1, -1), Wo3, bo3.reshape(1, 1))
    return out.reshape(-1)
